# baseline (speedup 1.0000x reference)
import functools

import jax
import jax.numpy as jnp
from jax import lax
from jax.experimental import pallas as pl
from jax.experimental.pallas import tpu as pltpu
from jax.experimental.pallas import tpu_sc as plsc

VOCAB = 1000000
EMBED_DIM = 32
BATCH = 4096
HIST_LEN = 50
TOTAL = BATCH * HIST_LEN
CHUNK = 128
NC = 2
NS = 16
NW = NC * NS
ROWS_PER_W = TOTAL // NW
N_CHUNKS = ROWS_PER_W // CHUNK


@functools.cache
def _make_kernel():
    mesh = plsc.VectorSubcoreMesh(core_axis_name="c", subcore_axis_name="s")

    @functools.partial(
        pl.kernel,
        mesh=mesh,
        out_type=jax.ShapeDtypeStruct((TOTAL, EMBED_DIM), jnp.float32),
        scratch_types=[
            pltpu.VMEM((N_CHUNKS, CHUNK), jnp.int32),
            pltpu.VMEM((CHUNK, EMBED_DIM), jnp.float32),
            pltpu.SemaphoreType.DMA,
        ],
        compiler_params=pltpu.CompilerParams(use_tc_tiling_on_sc=False),
    )
    def emb_kernel(table_hbm, idx_hbm, out_hbm, idx_v, rows_v, sem):
        wid = lax.axis_index("s") * NC + lax.axis_index("c")
        pltpu.sync_copy(idx_hbm.at[wid], idx_v)
        base = wid * ROWS_PER_W

        def body(j, carry):
            pltpu.async_copy(table_hbm.at[idx_v.at[j]], rows_v, sem).wait()
            pltpu.sync_copy(rows_v, out_hbm.at[pl.ds(base + j * CHUNK, CHUNK)])
            return carry

        lax.fori_loop(0, N_CHUNKS, body, 0)

    return emb_kernel


def kernel(input, emb_table):
    idx = input.astype(jnp.int32).reshape(NW, N_CHUNKS, CHUNK)
    out = _make_kernel()(emb_table, idx)
    return out.reshape(BATCH, HIST_LEN, EMBED_DIM)


# speedup vs baseline: 1.0383x; 1.0383x over previous
import functools

import jax
import jax.numpy as jnp
from jax import lax
from jax.experimental import pallas as pl
from jax.experimental.pallas import tpu as pltpu
from jax.experimental.pallas import tpu_sc as plsc

VOCAB = 1000000
EMBED_DIM = 32
BATCH = 4096
HIST_LEN = 50
TOTAL = BATCH * HIST_LEN
CHUNK = 128
NC = 2
NS = 16
NW = NC * NS
ROWS_PER_W = TOTAL // NW
N_CHUNKS = ROWS_PER_W // CHUNK
BLK_CHUNKS = 5
BLK_ROWS = BLK_CHUNKS * CHUNK
N_BLOCKS = N_CHUNKS // BLK_CHUNKS


@functools.cache
def _make_kernel():
    mesh = plsc.VectorSubcoreMesh(core_axis_name="c", subcore_axis_name="s")

    @functools.partial(
        pl.kernel,
        mesh=mesh,
        out_type=jax.ShapeDtypeStruct((TOTAL, EMBED_DIM), jnp.float32),
        scratch_types=[
            pltpu.VMEM((N_CHUNKS, CHUNK), jnp.int32),
            pltpu.VMEM((BLK_ROWS, EMBED_DIM), jnp.float32),
            pltpu.VMEM((BLK_ROWS, EMBED_DIM), jnp.float32),
            pltpu.SemaphoreType.DMA,
            pltpu.SemaphoreType.DMA,
            pltpu.SemaphoreType.DMA,
            pltpu.SemaphoreType.DMA,
        ],
        compiler_params=pltpu.CompilerParams(use_tc_tiling_on_sc=False),
    )
    def emb_kernel(table_hbm, idx_hbm, out_hbm, idx_v, buf0, buf1,
                   gsem0, gsem1, ssem0, ssem1):
        bufs = (buf0, buf1)
        gsems = (gsem0, gsem1)
        ssems = (ssem0, ssem1)
        wid = lax.axis_index("s") * NC + lax.axis_index("c")
        pltpu.sync_copy(idx_hbm.at[wid], idx_v)
        base = wid * ROWS_PER_W

        def fire_gathers(blk, b):
            for c in range(BLK_CHUNKS):
                pltpu.async_copy(
                    table_hbm.at[idx_v.at[blk * BLK_CHUNKS + c]],
                    bufs[b].at[pl.ds(c * CHUNK, CHUNK)],
                    gsems[b],
                )

        def drain_gathers(b):
            pltpu.make_async_copy(
                table_hbm.at[pl.ds(0, BLK_ROWS)], bufs[b], gsems[b]
            ).wait()

        def fire_store(blk, b):
            pltpu.async_copy(
                bufs[b], out_hbm.at[pl.ds(base + blk * BLK_ROWS, BLK_ROWS)],
                ssems[b],
            )

        def wait_store(b):
            pltpu.make_async_copy(
                bufs[b], out_hbm.at[pl.ds(base, BLK_ROWS)], ssems[b]
            ).wait()

        fire_gathers(0, 0)
        drain_gathers(0)
        fire_gathers(1, 1)
        fire_store(0, 0)

        @pl.loop(1, N_BLOCKS - 1, step=2)
        def _steady(o):
            for i in range(2):
                blk = o + i
                b = (1 + i) % 2
                drain_gathers(b)
                wait_store(1 - b)
                fire_gathers(blk + 1, 1 - b)
                fire_store(blk, b)

        drain_gathers(1)
        fire_store(N_BLOCKS - 1, 1)
        wait_store(0)
        wait_store(1)

    return emb_kernel


def kernel(input, emb_table):
    idx = input.astype(jnp.int32).reshape(NW, N_CHUNKS, CHUNK)
    out = _make_kernel()(emb_table, idx)
    return out.reshape(BATCH, HIST_LEN, EMBED_DIM)
